# baseline (speedup 1.0000x reference)
import functools

import jax
import jax.numpy as jnp
from jax import lax
from jax.experimental import pallas as pl
from jax.experimental.pallas import tpu as pltpu
from jax.experimental.pallas import tpu_sc as plsc

N = 10000
E = 320000
NGE = 160000
NDE = E - NGE
IN = 128
HID = 128

NP = 10240
DUMMY = 10000
NC, NS = 2, 16
BATCH = 128
NB = 160
CHB = 16
NCH = NB // CHB
EBATCH = 128
ENB = 160
EPT = NB * BATCH
EP = EPT * NS
RPT = NP // NS
NBLK = NP // 256

_MESH = plsc.VectorSubcoreMesh(
    core_axis_name="c", subcore_axis_name="s", num_cores=NC, num_subcores=NS)


def _make_edge_pass(np_=NP, ns=NS, nb=ENB, chb=CHB, batch=EBATCH, rpt=RPT,
                    interpret=False):
    nch = nb // chb
    nbuf = 2

    def body(srci_hbm, dsti_hbm, w_hbm, table, zeros,
             out, accum, srci, dsti, wv, r0b, r1b,
             g0, g1, s0, s1):
        bufs = [r0b, r1b]
        gsems = [g0, g1]
        ssems = [s0, s1]
        cid = lax.axis_index("c")
        sid = lax.axis_index("s")
        wid = cid * ns + sid
        r0 = sid * rpt
        pltpu.sync_copy(zeros.at[pl.ds(r0, rpt)], accum.at[pl.ds(r0, rpt)])
        plsc.subcore_barrier()

        def mult(rows, b):
            def group(g, c):
                wvec = wv[b, pl.ds(g * 16, 16)]
                for k16 in range(16):
                    k = g * 16 + k16
                    wk = wvec[k16]
                    for j in range(8):
                        sl = pl.ds(j * 16, 16)
                        rows[k, sl] = rows[k, sl] * wk
                return c

            lax.fori_loop(0, batch // 16, group, 0)

        def chunk(ch, carry):
            b0c = ch * chb
            pltpu.sync_copy(srci_hbm.at[wid, pl.ds(b0c, chb)], srci)
            pltpu.sync_copy(dsti_hbm.at[wid, pl.ds(b0c, chb)], dsti)
            pltpu.sync_copy(w_hbm.at[wid, pl.ds(b0c, chb)], wv)
            for t in range(nbuf):
                pltpu.async_copy(table.at[srci.at[t]], bufs[t], gsems[t])

            def pair(q, carry2):
                b0 = nbuf * q
                for t in range(nbuf):
                    bt = b0 + t
                    pltpu.make_async_copy(table.at[srci.at[bt]], bufs[t],
                                          gsems[t]).wait()
                    mult(bufs[t], bt)
                    pltpu.async_copy(bufs[t], accum.at[dsti.at[bt]],
                                     ssems[t], add=True)
                for t in range(nbuf):
                    bt = b0 + t
                    pltpu.make_async_copy(bufs[t], accum.at[dsti.at[bt]],
                                          ssems[t]).wait()

                    @pl.when(bt + nbuf < chb)
                    def _():
                        pltpu.async_copy(table.at[srci.at[bt + nbuf]],
                                         bufs[t], gsems[t])

                return carry2

            lax.fori_loop(0, chb // nbuf, pair, 0)
            return carry

        lax.fori_loop(0, nch, chunk, 0)
        plsc.subcore_barrier()
        pltpu.sync_copy(accum.at[pl.ds(r0, rpt)], out.at[cid, pl.ds(r0, rpt)])

    return pl.kernel(
        body,
        out_type=[jax.ShapeDtypeStruct((NC, np_, 128), jnp.float32)],
        mesh=plsc.VectorSubcoreMesh(core_axis_name="c", subcore_axis_name="s",
                                    num_cores=NC, num_subcores=ns),
        scratch_types=[
            pltpu.VMEM_SHARED((np_, 128), jnp.float32),
            pltpu.VMEM((chb, batch), jnp.int32),
            pltpu.VMEM((chb, batch), jnp.int32),
            pltpu.VMEM((chb, batch), jnp.float32),
            pltpu.VMEM((batch, 128), jnp.float32),
            pltpu.VMEM((batch, 128), jnp.float32),
        ] + [pltpu.SemaphoreType.DMA] * 4,
        interpret=interpret,
    )


def _make_deg_pass(np_=NP, ns=NS, nb=NB, chb=CHB, batch=BATCH, rpt=RPT,
                   interpret=False):
    nch = nb // chb

    def body(dsti_hbm, ones_hbm, zeros, out, accum, dsti, rows,
             sem_a, sem_b):
        cid = lax.axis_index("c")
        sid = lax.axis_index("s")
        wid = cid * ns + sid
        r0 = sid * rpt
        pltpu.sync_copy(zeros.at[pl.ds(r0, rpt)], accum.at[pl.ds(r0, rpt)])
        pltpu.sync_copy(ones_hbm, rows)
        plsc.subcore_barrier()

        def chunk(ch, carry):
            b0 = ch * chb
            pltpu.sync_copy(dsti_hbm.at[wid, pl.ds(b0, chb)], dsti)

            def pair(p, carry2):
                ba = 2 * p
                bb = 2 * p + 1
                pltpu.async_copy(rows, accum.at[dsti.at[ba]], sem_a,
                                 add=True)
                pltpu.async_copy(rows, accum.at[dsti.at[bb]], sem_b,
                                 add=True)
                pltpu.make_async_copy(rows, accum.at[dsti.at[ba]],
                                      sem_a).wait()
                pltpu.make_async_copy(rows, accum.at[dsti.at[bb]],
                                      sem_b).wait()
                return carry2

            lax.fori_loop(0, chb // 2, pair, 0)
            return carry

        lax.fori_loop(0, nch, chunk, 0)
        plsc.subcore_barrier()
        pltpu.sync_copy(accum.at[pl.ds(r0, rpt)], out.at[cid, pl.ds(r0, rpt)])

    return pl.kernel(
        body,
        out_type=[jax.ShapeDtypeStruct((NC, np_, 128), jnp.float32)],
        mesh=plsc.VectorSubcoreMesh(core_axis_name="c", subcore_axis_name="s",
                                    num_cores=NC, num_subcores=ns),
        scratch_types=[
            pltpu.VMEM_SHARED((np_, 128), jnp.float32),
            pltpu.VMEM((chb, batch), jnp.int32),
            pltpu.VMEM((batch, 128), jnp.float32),
            pltpu.SemaphoreType.DMA,
            pltpu.SemaphoreType.DMA,
        ],
        interpret=interpret,
    )


_edge_pass = _make_edge_pass()
_deg_pass = _make_deg_pass()


def _proj_body(x_ref, wu_ref, wd_ref, wb_ref, up_ref, dn_ref, bs_ref):
    x = x_ref[...]
    up_ref[...] = jnp.dot(x, wu_ref[...], preferred_element_type=jnp.float32)
    dn_ref[...] = jnp.dot(x, wd_ref[...], preferred_element_type=jnp.float32)
    bs_ref[...] = jnp.dot(x, wb_ref[...], preferred_element_type=jnp.float32)


def _make_proj(cin):
    return pl.pallas_call(
        _proj_body,
        grid=(NBLK,),
        in_specs=[
            pl.BlockSpec((256, cin), lambda i: (i, 0)),
            pl.BlockSpec((cin, 128), lambda i: (0, 0)),
            pl.BlockSpec((cin, 128), lambda i: (0, 0)),
            pl.BlockSpec((cin, 128), lambda i: (0, 0)),
        ],
        out_specs=[pl.BlockSpec((256, 128), lambda i: (i, 0))] * 3,
        out_shape=[jax.ShapeDtypeStruct((NP, 128), jnp.float32)] * 3,
    )


_proj0 = _make_proj(IN)


def _scaled_norm_h(u_ref, d_ref, b_ref, degu_ref, degd_ref):
    degu = degu_ref[...][:, 0:1]
    degd = degd_ref[...][:, 0:1]
    invu = jnp.where(degu > 0, 1.0 / degu, 0.0)
    invd = jnp.where(degd > 0, 1.0 / degd, 0.0)
    u = u_ref[...] * invu
    d = d_ref[...] * invd
    b = b_ref[...]
    ss = (jnp.sum(u * u, axis=1, keepdims=True)
          + jnp.sum(d * d, axis=1, keepdims=True)
          + jnp.sum(b * b, axis=1, keepdims=True))
    scale = 1.0 / jnp.maximum(jnp.sqrt(ss), 1e-12)
    h = jnp.concatenate([u, d, b], axis=1) * scale
    return jnp.where(h >= 0, h, 0.1 * h)


def _fuse_body(u_ref, d_ref, b_ref, degu_ref, degd_ref,
               wu_ref, wd_ref, wb_ref, up_ref, dn_ref, bs_ref):
    h = _scaled_norm_h(u_ref, d_ref, b_ref, degu_ref, degd_ref)
    up_ref[...] = jnp.dot(h, wu_ref[...], preferred_element_type=jnp.float32)
    dn_ref[...] = jnp.dot(h, wd_ref[...], preferred_element_type=jnp.float32)
    bs_ref[...] = jnp.dot(h, wb_ref[...], preferred_element_type=jnp.float32)


_fuse_proj = pl.pallas_call(
    _fuse_body,
    grid=(NBLK,),
    in_specs=[
        pl.BlockSpec((256, 128), lambda i: (i, 0)),
        pl.BlockSpec((256, 128), lambda i: (i, 0)),
        pl.BlockSpec((256, 128), lambda i: (i, 0)),
        pl.BlockSpec((256, 16), lambda i: (i, 0)),
        pl.BlockSpec((256, 16), lambda i: (i, 0)),
        pl.BlockSpec((384, 128), lambda i: (0, 0)),
        pl.BlockSpec((384, 128), lambda i: (0, 0)),
        pl.BlockSpec((384, 128), lambda i: (0, 0)),
    ],
    out_specs=[pl.BlockSpec((256, 128), lambda i: (i, 0))] * 3,
    out_shape=[jax.ShapeDtypeStruct((NP, 128), jnp.float32)] * 3,
)


def _final_body(u_ref, d_ref, b_ref, degu_ref, degd_ref, h_ref):
    h_ref[...] = _scaled_norm_h(u_ref, d_ref, b_ref, degu_ref, degd_ref)


_final = pl.pallas_call(
    _final_body,
    grid=(NBLK,),
    in_specs=[
        pl.BlockSpec((256, 128), lambda i: (i, 0)),
        pl.BlockSpec((256, 128), lambda i: (i, 0)),
        pl.BlockSpec((256, 128), lambda i: (i, 0)),
        pl.BlockSpec((256, 16), lambda i: (i, 0)),
        pl.BlockSpec((256, 16), lambda i: (i, 0)),
    ],
    out_specs=pl.BlockSpec((256, 384), lambda i: (i, 0)),
    out_shape=jax.ShapeDtypeStruct((NP, 384), jnp.float32),
)


def _mean_body(a0, a1, a2, b0, b1, b2, mu, md):
    mu[...] = (a0[...] + a1[...] + a2[...]) * (1.0 / 3.0)
    md[...] = (b0[...] + b1[...] + b2[...]) * (1.0 / 3.0)


_GW = 1280
_mean = pl.pallas_call(
    _mean_body,
    out_shape=[jax.ShapeDtypeStruct((_GW, 128), jnp.float32)] * 2,
)


def kernel(x, edge_index,
           Wup0, Wdown0, Wbias0, ugw0, dgw0,
           Wup1, Wdown1, Wbias1, ugw1, dgw1,
           Wup2, Wdown2, Wbias2, ugw2, dgw2):
    f32 = jnp.float32
    ei = edge_index.astype(jnp.int32)
    row, col = ei[0], ei[1]
    pad = EP - E
    padi = jnp.full((pad,), DUMMY, jnp.int32)
    src = jnp.stack([
        jnp.concatenate([row, jnp.zeros((pad,), jnp.int32)]),
        jnp.concatenate([col + NP, jnp.full((pad,), NP, jnp.int32)]),
    ]).reshape(NC * NS, NB, BATCH)
    dst = jnp.stack([
        jnp.concatenate([col, padi]),
        jnp.concatenate([row, padi]),
    ]).reshape(NC * NS, NB, BATCH)

    src_e = src.reshape(NC * NS, ENB, EBATCH)
    dst_e = dst.reshape(NC * NS, ENB, EBATCH)

    ones_nde = jnp.ones((NDE,), f32)
    padw = jnp.zeros((pad,), f32)

    def edge_w(ug, dg):
        return jnp.stack([
            jnp.concatenate([ug, ones_nde, padw]),
            jnp.concatenate([dg, ones_nde, padw]),
        ]).reshape(NC * NS, ENB, EBATCH)

    zeros = jnp.zeros((NP, 128), f32)

    xp = jnp.pad(x, ((0, NP - N), (0, 0)))

    (deg,) = _deg_pass(dst, jnp.ones((BATCH, 128), f32), zeros)
    degu, degd = deg[0, :, :16], deg[1, :, :16]

    up0, dn0, bs0 = _proj0(xp, Wup0.T, Wdown0.T, Wbias0.T)
    tbl0 = jnp.concatenate([up0, dn0], axis=0)
    (sums0,) = _edge_pass(src_e, dst_e, edge_w(ugw0, dgw0), tbl0, zeros)

    up1, dn1, bs1 = _fuse_proj(sums0[0], sums0[1], bs0, degu, degd,
                               Wup1.T, Wdown1.T, Wbias1.T)
    tbl1 = jnp.concatenate([up1, dn1], axis=0)
    (sums1,) = _edge_pass(src_e, dst_e, edge_w(ugw1, dgw1), tbl1, zeros)

    up2, dn2, bs2 = _fuse_proj(sums1[0], sums1[1], bs1, degu, degd,
                               Wup2.T, Wdown2.T, Wbias2.T)
    tbl2 = jnp.concatenate([up2, dn2], axis=0)
    (sums2,) = _edge_pass(src_e, dst_e, edge_w(ugw2, dgw2), tbl2, zeros)

    h = _final(sums2[0], sums2[1], bs2, degu, degd)[:N]

    gpad = _GW * 128 - NGE
    def gw_pad(a):
        return jnp.pad(a, (0, gpad)).reshape(_GW, 128)
    mu, md = _mean(gw_pad(ugw0), gw_pad(ugw1), gw_pad(ugw2),
                   gw_pad(dgw0), gw_pad(dgw1), gw_pad(dgw2))
    mean_up = jnp.concatenate([mu.reshape(-1)[:NGE], ones_nde])
    mean_down = jnp.concatenate([md.reshape(-1)[:NGE], ones_nde])
    return h, mean_up, mean_down


# speedup vs baseline: 1.0458x; 1.0458x over previous
import functools

import jax
import jax.numpy as jnp
from jax import lax
from jax.experimental import pallas as pl
from jax.experimental.pallas import tpu as pltpu
from jax.experimental.pallas import tpu_sc as plsc

N = 10000
E = 320000
NGE = 160000
NDE = E - NGE
IN = 128
HID = 128

NP = 10240
DUMMY = 10000
NC, NS = 2, 16
BATCH = 128
NB = 160
CHB = 16
NCH = NB // CHB
EBATCH = 128
ENB = 160
EPT = NB * BATCH
EP = EPT * NS
RPT = NP // NS
NBLK = NP // 256

_MESH = plsc.VectorSubcoreMesh(
    core_axis_name="c", subcore_axis_name="s", num_cores=NC, num_subcores=NS)


def _make_edge_pass(np_=NP, ns=NS, nb=ENB, chb=32, batch=EBATCH, rpt=RPT,
                    interpret=False):
    nch = nb // chb
    nbuf = 2

    def body(srci_hbm, dsti_hbm, w_hbm, table, zeros,
             out, accum, srci, dsti, wv, r0b, r1b,
             g0, g1, s0, s1):
        bufs = [r0b, r1b]
        gsems = [g0, g1]
        ssems = [s0, s1]
        cid = lax.axis_index("c")
        sid = lax.axis_index("s")
        wid = cid * ns + sid
        r0 = sid * rpt
        pltpu.sync_copy(zeros.at[pl.ds(r0, rpt)], accum.at[pl.ds(r0, rpt)])
        plsc.subcore_barrier()

        def mult(rows, b):
            def group(g, c):
                wvec = wv[b, pl.ds(g * 16, 16)]
                for k16 in range(16):
                    k = g * 16 + k16
                    wk = wvec[k16]
                    for j in range(8):
                        sl = pl.ds(j * 16, 16)
                        rows[k, sl] = rows[k, sl] * wk
                return c

            lax.fori_loop(0, batch // 16, group, 0)

        def chunk(ch, carry):
            b0c = ch * chb
            pltpu.sync_copy(srci_hbm.at[wid, pl.ds(b0c, chb)], srci)
            pltpu.sync_copy(dsti_hbm.at[wid, pl.ds(b0c, chb)], dsti)
            pltpu.sync_copy(w_hbm.at[wid, pl.ds(b0c, chb)], wv)
            rows_a, rows_b = bufs
            sem_ga, sem_gb = gsems
            sem_sa, sem_sb = ssems
            pltpu.async_copy(table.at[srci.at[0]], rows_a, sem_ga)

            def pair(p, carry2):
                ba = 2 * p
                bb = 2 * p + 1
                pltpu.make_async_copy(table.at[srci.at[ba]], rows_a,
                                      sem_ga).wait()
                pltpu.async_copy(table.at[srci.at[bb]], rows_b, sem_gb)
                mult(rows_a, ba)
                pltpu.async_copy(rows_a, accum.at[dsti.at[ba]], sem_sa,
                                 add=True)
                pltpu.make_async_copy(table.at[srci.at[bb]], rows_b,
                                      sem_gb).wait()
                pltpu.make_async_copy(rows_a, accum.at[dsti.at[ba]],
                                      sem_sa).wait()

                @pl.when(ba + 2 < chb)
                def _():
                    pltpu.async_copy(table.at[srci.at[ba + 2]], rows_a,
                                     sem_ga)

                mult(rows_b, bb)
                pltpu.async_copy(rows_b, accum.at[dsti.at[bb]], sem_sb,
                                 add=True)
                pltpu.make_async_copy(rows_b, accum.at[dsti.at[bb]],
                                      sem_sb).wait()
                return carry2

            lax.fori_loop(0, chb // nbuf, pair, 0)
            return carry

        lax.fori_loop(0, nch, chunk, 0)
        plsc.subcore_barrier()
        pltpu.sync_copy(accum.at[pl.ds(r0, rpt)], out.at[cid, pl.ds(r0, rpt)])

    return pl.kernel(
        body,
        out_type=[jax.ShapeDtypeStruct((NC, np_, 128), jnp.float32)],
        mesh=plsc.VectorSubcoreMesh(core_axis_name="c", subcore_axis_name="s",
                                    num_cores=NC, num_subcores=ns),
        scratch_types=[
            pltpu.VMEM_SHARED((np_, 128), jnp.float32),
            pltpu.VMEM((chb, batch), jnp.int32),
            pltpu.VMEM((chb, batch), jnp.int32),
            pltpu.VMEM((chb, batch), jnp.float32),
            pltpu.VMEM((batch, 128), jnp.float32),
            pltpu.VMEM((batch, 128), jnp.float32),
        ] + [pltpu.SemaphoreType.DMA] * 4,
        interpret=interpret,
    )


def _make_deg_pass(np_=NP, ns=NS, nb=NB, chb=CHB, batch=BATCH, rpt=RPT,
                   interpret=False):
    nch = nb // chb

    def body(dsti_hbm, ones_hbm, zeros, out, accum, dsti, rows,
             sem_a, sem_b):
        cid = lax.axis_index("c")
        sid = lax.axis_index("s")
        wid = cid * ns + sid
        r0 = sid * rpt
        pltpu.sync_copy(zeros.at[pl.ds(r0, rpt)], accum.at[pl.ds(r0, rpt)])
        pltpu.sync_copy(ones_hbm, rows)
        plsc.subcore_barrier()

        def chunk(ch, carry):
            b0 = ch * chb
            pltpu.sync_copy(dsti_hbm.at[wid, pl.ds(b0, chb)], dsti)

            def pair(p, carry2):
                ba = 2 * p
                bb = 2 * p + 1
                pltpu.async_copy(rows, accum.at[dsti.at[ba]], sem_a,
                                 add=True)
                pltpu.async_copy(rows, accum.at[dsti.at[bb]], sem_b,
                                 add=True)
                pltpu.make_async_copy(rows, accum.at[dsti.at[ba]],
                                      sem_a).wait()
                pltpu.make_async_copy(rows, accum.at[dsti.at[bb]],
                                      sem_b).wait()
                return carry2

            lax.fori_loop(0, chb // 2, pair, 0)
            return carry

        lax.fori_loop(0, nch, chunk, 0)
        plsc.subcore_barrier()
        pltpu.sync_copy(accum.at[pl.ds(r0, rpt)], out.at[cid, pl.ds(r0, rpt)])

    return pl.kernel(
        body,
        out_type=[jax.ShapeDtypeStruct((NC, np_, 128), jnp.float32)],
        mesh=plsc.VectorSubcoreMesh(core_axis_name="c", subcore_axis_name="s",
                                    num_cores=NC, num_subcores=ns),
        scratch_types=[
            pltpu.VMEM_SHARED((np_, 128), jnp.float32),
            pltpu.VMEM((chb, batch), jnp.int32),
            pltpu.VMEM((batch, 128), jnp.float32),
            pltpu.SemaphoreType.DMA,
            pltpu.SemaphoreType.DMA,
        ],
        interpret=interpret,
    )


_edge_pass = _make_edge_pass()
_deg_pass = _make_deg_pass()


def _proj_body(x_ref, wu_ref, wd_ref, wb_ref, up_ref, dn_ref, bs_ref):
    x = x_ref[...]
    up_ref[...] = jnp.dot(x, wu_ref[...], preferred_element_type=jnp.float32)
    dn_ref[...] = jnp.dot(x, wd_ref[...], preferred_element_type=jnp.float32)
    bs_ref[...] = jnp.dot(x, wb_ref[...], preferred_element_type=jnp.float32)


def _make_proj(cin):
    return pl.pallas_call(
        _proj_body,
        grid=(NBLK,),
        in_specs=[
            pl.BlockSpec((256, cin), lambda i: (i, 0)),
            pl.BlockSpec((cin, 128), lambda i: (0, 0)),
            pl.BlockSpec((cin, 128), lambda i: (0, 0)),
            pl.BlockSpec((cin, 128), lambda i: (0, 0)),
        ],
        out_specs=[pl.BlockSpec((256, 128), lambda i: (i, 0))] * 3,
        out_shape=[jax.ShapeDtypeStruct((NP, 128), jnp.float32)] * 3,
    )


_proj0 = _make_proj(IN)


def _scaled_norm_h(u_ref, d_ref, b_ref, degu_ref, degd_ref):
    degu = degu_ref[...][:, 0:1]
    degd = degd_ref[...][:, 0:1]
    invu = jnp.where(degu > 0, 1.0 / degu, 0.0)
    invd = jnp.where(degd > 0, 1.0 / degd, 0.0)
    u = u_ref[...] * invu
    d = d_ref[...] * invd
    b = b_ref[...]
    ss = (jnp.sum(u * u, axis=1, keepdims=True)
          + jnp.sum(d * d, axis=1, keepdims=True)
          + jnp.sum(b * b, axis=1, keepdims=True))
    scale = 1.0 / jnp.maximum(jnp.sqrt(ss), 1e-12)
    h = jnp.concatenate([u, d, b], axis=1) * scale
    return jnp.where(h >= 0, h, 0.1 * h)


def _fuse_body(u_ref, d_ref, b_ref, degu_ref, degd_ref,
               wu_ref, wd_ref, wb_ref, up_ref, dn_ref, bs_ref):
    h = _scaled_norm_h(u_ref, d_ref, b_ref, degu_ref, degd_ref)
    up_ref[...] = jnp.dot(h, wu_ref[...], preferred_element_type=jnp.float32)
    dn_ref[...] = jnp.dot(h, wd_ref[...], preferred_element_type=jnp.float32)
    bs_ref[...] = jnp.dot(h, wb_ref[...], preferred_element_type=jnp.float32)


_fuse_proj = pl.pallas_call(
    _fuse_body,
    grid=(NBLK,),
    in_specs=[
        pl.BlockSpec((256, 128), lambda i: (i, 0)),
        pl.BlockSpec((256, 128), lambda i: (i, 0)),
        pl.BlockSpec((256, 128), lambda i: (i, 0)),
        pl.BlockSpec((256, 16), lambda i: (i, 0)),
        pl.BlockSpec((256, 16), lambda i: (i, 0)),
        pl.BlockSpec((384, 128), lambda i: (0, 0)),
        pl.BlockSpec((384, 128), lambda i: (0, 0)),
        pl.BlockSpec((384, 128), lambda i: (0, 0)),
    ],
    out_specs=[pl.BlockSpec((256, 128), lambda i: (i, 0))] * 3,
    out_shape=[jax.ShapeDtypeStruct((NP, 128), jnp.float32)] * 3,
)


def _final_body(u_ref, d_ref, b_ref, degu_ref, degd_ref, h_ref):
    h_ref[...] = _scaled_norm_h(u_ref, d_ref, b_ref, degu_ref, degd_ref)


_final = pl.pallas_call(
    _final_body,
    grid=(NBLK,),
    in_specs=[
        pl.BlockSpec((256, 128), lambda i: (i, 0)),
        pl.BlockSpec((256, 128), lambda i: (i, 0)),
        pl.BlockSpec((256, 128), lambda i: (i, 0)),
        pl.BlockSpec((256, 16), lambda i: (i, 0)),
        pl.BlockSpec((256, 16), lambda i: (i, 0)),
    ],
    out_specs=pl.BlockSpec((256, 384), lambda i: (i, 0)),
    out_shape=jax.ShapeDtypeStruct((NP, 384), jnp.float32),
)


def _mean_body(a0, a1, a2, b0, b1, b2, mu, md):
    mu[...] = (a0[...] + a1[...] + a2[...]) * (1.0 / 3.0)
    md[...] = (b0[...] + b1[...] + b2[...]) * (1.0 / 3.0)


_GW = 1280
_mean = pl.pallas_call(
    _mean_body,
    out_shape=[jax.ShapeDtypeStruct((_GW, 128), jnp.float32)] * 2,
)


def kernel(x, edge_index,
           Wup0, Wdown0, Wbias0, ugw0, dgw0,
           Wup1, Wdown1, Wbias1, ugw1, dgw1,
           Wup2, Wdown2, Wbias2, ugw2, dgw2):
    f32 = jnp.float32
    ei = edge_index.astype(jnp.int32)
    row, col = ei[0], ei[1]
    pad = EP - E
    padi = jnp.full((pad,), DUMMY, jnp.int32)
    src = jnp.stack([
        jnp.concatenate([row, jnp.zeros((pad,), jnp.int32)]),
        jnp.concatenate([col + NP, jnp.full((pad,), NP, jnp.int32)]),
    ]).reshape(NC * NS, NB, BATCH)
    dst = jnp.stack([
        jnp.concatenate([col, padi]),
        jnp.concatenate([row, padi]),
    ]).reshape(NC * NS, NB, BATCH)

    src_e = src.reshape(NC * NS, ENB, EBATCH)
    dst_e = dst.reshape(NC * NS, ENB, EBATCH)

    ones_nde = jnp.ones((NDE,), f32)
    padw = jnp.zeros((pad,), f32)

    def edge_w(ug, dg):
        return jnp.stack([
            jnp.concatenate([ug, ones_nde, padw]),
            jnp.concatenate([dg, ones_nde, padw]),
        ]).reshape(NC * NS, ENB, EBATCH)

    zeros = jnp.zeros((NP, 128), f32)

    xp = jnp.pad(x, ((0, NP - N), (0, 0)))

    (deg,) = _deg_pass(dst, jnp.ones((BATCH, 128), f32), zeros)
    degu, degd = deg[0, :, :16], deg[1, :, :16]

    up0, dn0, bs0 = _proj0(xp, Wup0.T, Wdown0.T, Wbias0.T)
    tbl0 = jnp.concatenate([up0, dn0], axis=0)
    (sums0,) = _edge_pass(src_e, dst_e, edge_w(ugw0, dgw0), tbl0, zeros)

    up1, dn1, bs1 = _fuse_proj(sums0[0], sums0[1], bs0, degu, degd,
                               Wup1.T, Wdown1.T, Wbias1.T)
    tbl1 = jnp.concatenate([up1, dn1], axis=0)
    (sums1,) = _edge_pass(src_e, dst_e, edge_w(ugw1, dgw1), tbl1, zeros)

    up2, dn2, bs2 = _fuse_proj(sums1[0], sums1[1], bs1, degu, degd,
                               Wup2.T, Wdown2.T, Wbias2.T)
    tbl2 = jnp.concatenate([up2, dn2], axis=0)
    (sums2,) = _edge_pass(src_e, dst_e, edge_w(ugw2, dgw2), tbl2, zeros)

    h = _final(sums2[0], sums2[1], bs2, degu, degd)[:N]

    gpad = _GW * 128 - NGE
    def gw_pad(a):
        return jnp.pad(a, (0, gpad)).reshape(_GW, 128)
    mu, md = _mean(gw_pad(ugw0), gw_pad(ugw1), gw_pad(ugw2),
                   gw_pad(dgw0), gw_pad(dgw1), gw_pad(dgw2))
    mean_up = jnp.concatenate([mu.reshape(-1)[:NGE], ones_nde])
    mean_down = jnp.concatenate([md.reshape(-1)[:NGE], ones_nde])
    return h, mean_up, mean_down


# speedup vs baseline: 1.0490x; 1.0031x over previous
import jax
import jax.numpy as jnp
from jax import lax
from jax.experimental import pallas as pl
from jax.experimental.pallas import tpu as pltpu
from jax.experimental.pallas import tpu_sc as plsc

N = 10000
E = 320000
NGE = 160000
NDE = E - NGE
IN = 128
HID = 128

NP = 10240
DUMMY = 10000
NC, NS = 2, 16
BATCH = 128
NB = 160
CHB = 16
NCH = NB // CHB
EBATCH = 128
ENB = 160
EPT = NB * BATCH
EP = EPT * NS
RPT = NP // NS
NBLK = NP // 256

_MESH = plsc.VectorSubcoreMesh(
    core_axis_name="c", subcore_axis_name="s", num_cores=NC, num_subcores=NS)


def _make_edge_pass(np_=NP, ns=NS, nb=ENB, chb=32, batch=EBATCH, rpt=RPT,
                    interpret=False):
    nch = nb // chb
    nbuf = 2

    def body(srci_hbm, dsti_hbm, w_hbm, table, zeros,
             out, accum, srci, dsti, wv, r0b, r1b,
             g0, g1, s0, s1):
        bufs = [r0b, r1b]
        gsems = [g0, g1]
        ssems = [s0, s1]
        cid = lax.axis_index("c")
        sid = lax.axis_index("s")
        wid = cid * ns + sid
        r0 = sid * rpt
        pltpu.sync_copy(zeros.at[pl.ds(r0, rpt)], accum.at[pl.ds(r0, rpt)])
        plsc.subcore_barrier()

        def mult(rows, b):
            def group(g, c):
                wvec = wv[b, pl.ds(g * 16, 16)]
                for k16 in range(16):
                    k = g * 16 + k16
                    wk = wvec[k16]
                    for j in range(8):
                        sl = pl.ds(j * 16, 16)
                        rows[k, sl] = rows[k, sl] * wk
                return c

            lax.fori_loop(0, batch // 16, group, 0)

        def chunk(ch, carry):
            b0c = ch * chb
            sem_st = gsems[1]
            pltpu.async_copy(srci_hbm.at[wid, pl.ds(b0c, chb)], srci, sem_st)
            pltpu.async_copy(dsti_hbm.at[wid, pl.ds(b0c, chb)], dsti, sem_st)
            pltpu.async_copy(w_hbm.at[wid, pl.ds(b0c, chb)], wv, sem_st)
            pltpu.make_async_copy(srci_hbm.at[wid, pl.ds(b0c, chb)], srci,
                                  sem_st).wait()
            pltpu.make_async_copy(dsti_hbm.at[wid, pl.ds(b0c, chb)], dsti,
                                  sem_st).wait()
            pltpu.make_async_copy(w_hbm.at[wid, pl.ds(b0c, chb)], wv,
                                  sem_st).wait()
            rows_a, rows_b = bufs
            sem_ga, sem_gb = gsems
            sem_sa, sem_sb = ssems
            pltpu.async_copy(table.at[srci.at[0]], rows_a, sem_ga)

            def pair(p, carry2):
                ba = 2 * p
                bb = 2 * p + 1
                pltpu.make_async_copy(table.at[srci.at[ba]], rows_a,
                                      sem_ga).wait()
                pltpu.async_copy(table.at[srci.at[bb]], rows_b, sem_gb)
                mult(rows_a, ba)
                pltpu.async_copy(rows_a, accum.at[dsti.at[ba]], sem_sa,
                                 add=True)
                pltpu.make_async_copy(table.at[srci.at[bb]], rows_b,
                                      sem_gb).wait()
                pltpu.make_async_copy(rows_a, accum.at[dsti.at[ba]],
                                      sem_sa).wait()

                @pl.when(ba + 2 < chb)
                def _():
                    pltpu.async_copy(table.at[srci.at[ba + 2]], rows_a,
                                     sem_ga)

                mult(rows_b, bb)
                pltpu.async_copy(rows_b, accum.at[dsti.at[bb]], sem_sb,
                                 add=True)
                pltpu.make_async_copy(rows_b, accum.at[dsti.at[bb]],
                                      sem_sb).wait()
                return carry2

            lax.fori_loop(0, chb // nbuf, pair, 0)
            return carry

        lax.fori_loop(0, nch, chunk, 0)
        plsc.subcore_barrier()
        pltpu.sync_copy(accum.at[pl.ds(r0, rpt)], out.at[cid, pl.ds(r0, rpt)])

    return pl.kernel(
        body,
        out_type=[jax.ShapeDtypeStruct((NC, np_, 128), jnp.float32)],
        mesh=plsc.VectorSubcoreMesh(core_axis_name="c", subcore_axis_name="s",
                                    num_cores=NC, num_subcores=ns),
        scratch_types=[
            pltpu.VMEM_SHARED((np_, 128), jnp.float32),
            pltpu.VMEM((chb, batch), jnp.int32),
            pltpu.VMEM((chb, batch), jnp.int32),
            pltpu.VMEM((chb, batch), jnp.float32),
            pltpu.VMEM((batch, 128), jnp.float32),
            pltpu.VMEM((batch, 128), jnp.float32),
        ] + [pltpu.SemaphoreType.DMA] * 4,
        interpret=interpret,
    )


def _make_deg_pass(np_=NP, ns=NS, nb=NB, chb=CHB, batch=BATCH, rpt=RPT,
                   interpret=False):
    nch = nb // chb

    def body(dsti_hbm, ones_hbm, zeros, out, accum, dsti, rows,
             sem_a, sem_b):
        cid = lax.axis_index("c")
        sid = lax.axis_index("s")
        wid = cid * ns + sid
        r0 = sid * rpt
        pltpu.sync_copy(zeros.at[pl.ds(r0, rpt)], accum.at[pl.ds(r0, rpt)])
        pltpu.sync_copy(ones_hbm, rows)
        plsc.subcore_barrier()

        def chunk(ch, carry):
            b0 = ch * chb
            pltpu.sync_copy(dsti_hbm.at[wid, pl.ds(b0, chb)], dsti)

            def pair(p, carry2):
                ba = 2 * p
                bb = 2 * p + 1
                pltpu.async_copy(rows, accum.at[dsti.at[ba]], sem_a,
                                 add=True)
                pltpu.async_copy(rows, accum.at[dsti.at[bb]], sem_b,
                                 add=True)
                pltpu.make_async_copy(rows, accum.at[dsti.at[ba]],
                                      sem_a).wait()
                pltpu.make_async_copy(rows, accum.at[dsti.at[bb]],
                                      sem_b).wait()
                return carry2

            lax.fori_loop(0, chb // 2, pair, 0)
            return carry

        lax.fori_loop(0, nch, chunk, 0)
        plsc.subcore_barrier()
        pltpu.sync_copy(accum.at[pl.ds(r0, rpt)], out.at[cid, pl.ds(r0, rpt)])

    return pl.kernel(
        body,
        out_type=[jax.ShapeDtypeStruct((NC, np_, 128), jnp.float32)],
        mesh=plsc.VectorSubcoreMesh(core_axis_name="c", subcore_axis_name="s",
                                    num_cores=NC, num_subcores=ns),
        scratch_types=[
            pltpu.VMEM_SHARED((np_, 128), jnp.float32),
            pltpu.VMEM((chb, batch), jnp.int32),
            pltpu.VMEM((batch, 128), jnp.float32),
            pltpu.SemaphoreType.DMA,
            pltpu.SemaphoreType.DMA,
        ],
        interpret=interpret,
    )


_edge_pass = _make_edge_pass()
_deg_pass = _make_deg_pass()


def _proj_body(x_ref, wu_ref, wd_ref, wb_ref, up_ref, dn_ref, bs_ref):
    x = x_ref[...]
    up_ref[...] = jnp.dot(x, wu_ref[...], preferred_element_type=jnp.float32)
    dn_ref[...] = jnp.dot(x, wd_ref[...], preferred_element_type=jnp.float32)
    bs_ref[...] = jnp.dot(x, wb_ref[...], preferred_element_type=jnp.float32)


def _make_proj(cin):
    return pl.pallas_call(
        _proj_body,
        grid=(NBLK,),
        in_specs=[
            pl.BlockSpec((256, cin), lambda i: (i, 0)),
            pl.BlockSpec((cin, 128), lambda i: (0, 0)),
            pl.BlockSpec((cin, 128), lambda i: (0, 0)),
            pl.BlockSpec((cin, 128), lambda i: (0, 0)),
        ],
        out_specs=[pl.BlockSpec((256, 128), lambda i: (i, 0))] * 3,
        out_shape=[jax.ShapeDtypeStruct((NP, 128), jnp.float32)] * 3,
    )


_proj0 = _make_proj(IN)


def _scaled_norm_h(u_ref, d_ref, b_ref, degu_ref, degd_ref):
    degu = degu_ref[...][:, 0:1]
    degd = degd_ref[...][:, 0:1]
    invu = jnp.where(degu > 0, 1.0 / degu, 0.0)
    invd = jnp.where(degd > 0, 1.0 / degd, 0.0)
    u = u_ref[...] * invu
    d = d_ref[...] * invd
    b = b_ref[...]
    ss = (jnp.sum(u * u, axis=1, keepdims=True)
          + jnp.sum(d * d, axis=1, keepdims=True)
          + jnp.sum(b * b, axis=1, keepdims=True))
    scale = 1.0 / jnp.maximum(jnp.sqrt(ss), 1e-12)
    h = jnp.concatenate([u, d, b], axis=1) * scale
    return jnp.where(h >= 0, h, 0.1 * h)


def _fuse_body(u_ref, d_ref, b_ref, degu_ref, degd_ref,
               wu_ref, wd_ref, wb_ref, up_ref, dn_ref, bs_ref):
    h = _scaled_norm_h(u_ref, d_ref, b_ref, degu_ref, degd_ref)
    up_ref[...] = jnp.dot(h, wu_ref[...], preferred_element_type=jnp.float32)
    dn_ref[...] = jnp.dot(h, wd_ref[...], preferred_element_type=jnp.float32)
    bs_ref[...] = jnp.dot(h, wb_ref[...], preferred_element_type=jnp.float32)


_fuse_proj = pl.pallas_call(
    _fuse_body,
    grid=(NBLK,),
    in_specs=[
        pl.BlockSpec((256, 128), lambda i: (i, 0)),
        pl.BlockSpec((256, 128), lambda i: (i, 0)),
        pl.BlockSpec((256, 128), lambda i: (i, 0)),
        pl.BlockSpec((256, 16), lambda i: (i, 0)),
        pl.BlockSpec((256, 16), lambda i: (i, 0)),
        pl.BlockSpec((384, 128), lambda i: (0, 0)),
        pl.BlockSpec((384, 128), lambda i: (0, 0)),
        pl.BlockSpec((384, 128), lambda i: (0, 0)),
    ],
    out_specs=[pl.BlockSpec((256, 128), lambda i: (i, 0))] * 3,
    out_shape=[jax.ShapeDtypeStruct((NP, 128), jnp.float32)] * 3,
)


def _final_body(u_ref, d_ref, b_ref, degu_ref, degd_ref, h_ref):
    h_ref[...] = _scaled_norm_h(u_ref, d_ref, b_ref, degu_ref, degd_ref)


_final = pl.pallas_call(
    _final_body,
    grid=(NBLK,),
    in_specs=[
        pl.BlockSpec((256, 128), lambda i: (i, 0)),
        pl.BlockSpec((256, 128), lambda i: (i, 0)),
        pl.BlockSpec((256, 128), lambda i: (i, 0)),
        pl.BlockSpec((256, 16), lambda i: (i, 0)),
        pl.BlockSpec((256, 16), lambda i: (i, 0)),
    ],
    out_specs=pl.BlockSpec((256, 384), lambda i: (i, 0)),
    out_shape=jax.ShapeDtypeStruct((NP, 384), jnp.float32),
)


def _mean_body(a0, a1, a2, b0, b1, b2, mu, md):
    mu[...] = (a0[...] + a1[...] + a2[...]) * (1.0 / 3.0)
    md[...] = (b0[...] + b1[...] + b2[...]) * (1.0 / 3.0)


_GW = 1280
_mean = pl.pallas_call(
    _mean_body,
    out_shape=[jax.ShapeDtypeStruct((_GW, 128), jnp.float32)] * 2,
)


def kernel(x, edge_index,
           Wup0, Wdown0, Wbias0, ugw0, dgw0,
           Wup1, Wdown1, Wbias1, ugw1, dgw1,
           Wup2, Wdown2, Wbias2, ugw2, dgw2):
    f32 = jnp.float32
    ei = edge_index.astype(jnp.int32)
    row, col = ei[0], ei[1]
    pad = EP - E
    padi = jnp.full((pad,), DUMMY, jnp.int32)
    src = jnp.stack([
        jnp.concatenate([row, jnp.zeros((pad,), jnp.int32)]),
        jnp.concatenate([col + NP, jnp.full((pad,), NP, jnp.int32)]),
    ]).reshape(NC * NS, NB, BATCH)
    dst = jnp.stack([
        jnp.concatenate([col, padi]),
        jnp.concatenate([row, padi]),
    ]).reshape(NC * NS, NB, BATCH)

    src_e = src.reshape(NC * NS, ENB, EBATCH)
    dst_e = dst.reshape(NC * NS, ENB, EBATCH)

    ones_nde = jnp.ones((NDE,), f32)
    padw = jnp.zeros((pad,), f32)

    def edge_w(ug, dg):
        return jnp.stack([
            jnp.concatenate([ug, ones_nde, padw]),
            jnp.concatenate([dg, ones_nde, padw]),
        ]).reshape(NC * NS, ENB, EBATCH)

    zeros = jnp.zeros((NP, 128), f32)

    xp = jnp.pad(x, ((0, NP - N), (0, 0)))

    (deg,) = _deg_pass(dst, jnp.ones((BATCH, 128), f32), zeros)
    degu, degd = deg[0, :, :16], deg[1, :, :16]

    up0, dn0, bs0 = _proj0(xp, Wup0.T, Wdown0.T, Wbias0.T)
    tbl0 = jnp.concatenate([up0, dn0], axis=0)
    (sums0,) = _edge_pass(src_e, dst_e, edge_w(ugw0, dgw0), tbl0, zeros)

    up1, dn1, bs1 = _fuse_proj(sums0[0], sums0[1], bs0, degu, degd,
                               Wup1.T, Wdown1.T, Wbias1.T)
    tbl1 = jnp.concatenate([up1, dn1], axis=0)
    (sums1,) = _edge_pass(src_e, dst_e, edge_w(ugw1, dgw1), tbl1, zeros)

    up2, dn2, bs2 = _fuse_proj(sums1[0], sums1[1], bs1, degu, degd,
                               Wup2.T, Wdown2.T, Wbias2.T)
    tbl2 = jnp.concatenate([up2, dn2], axis=0)
    (sums2,) = _edge_pass(src_e, dst_e, edge_w(ugw2, dgw2), tbl2, zeros)

    h = _final(sums2[0], sums2[1], bs2, degu, degd)[:N]

    gpad = _GW * 128 - NGE
    def gw_pad(a):
        return jnp.pad(a, (0, gpad)).reshape(_GW, 128)
    mu, md = _mean(gw_pad(ugw0), gw_pad(ugw1), gw_pad(ugw2),
                   gw_pad(dgw0), gw_pad(dgw1), gw_pad(dgw2))
    mean_up = jnp.concatenate([mu.reshape(-1)[:NGE], ones_nde])
    mean_down = jnp.concatenate([md.reshape(-1)[:NGE], ones_nde])
    return h, mean_up, mean_down
